# baseline (speedup 1.0000x reference)
import functools

import jax
import jax.numpy as jnp
from jax import lax
from jax.experimental import pallas as pl
from jax.experimental.pallas import tpu as pltpu
from jax.experimental.pallas import tpu_sc as plsc

NC = 2
NS = 16
LANES = 16
CHUNK = 128
_CORE0_FRAC = 0.25

_MESH = dict(core_axis_name="c", subcore_axis_name="s", num_cores=NC,
             num_subcores=NS)


def _vzero(ref, n):
    for k in range(n // LANES):
        ref[pl.ds(k * LANES, LANES)] = jnp.zeros((LANES,), jnp.float32)
    if n % LANES:
        ref[pl.ds(n - LANES, LANES)] = jnp.zeros((LANES,), jnp.float32)


@functools.partial(jax.jit, static_argnames=("np_", "kc"))
def _deg_call(dstp, np_, kc):
    slab = np_ // NS
    mesh = plsc.VectorSubcoreMesh(**_MESH)

    @functools.partial(
        pl.kernel,
        out_type=jax.ShapeDtypeStruct((NC * np_,), jnp.float32),
        mesh=mesh,
        scratch_types=[
            pltpu.VMEM((kc, CHUNK), jnp.int32),
            pltpu.VMEM((CHUNK,), jnp.float32),
            pltpu.VMEM((slab,), jnp.float32),
            pltpu.VMEM_SHARED((np_,), jnp.float32),
            pltpu.SemaphoreType.DMA,
        ],
    )
    def k(dst_hbm, out_hbm, idx_v, ones_v, zbuf, hist, sem):
        c = lax.axis_index("c")
        s = lax.axis_index("s")
        wid = s * NC + c
        base = s * slab
        _vzero(zbuf, slab)
        for kk in range(CHUNK // LANES):
            ones_v[pl.ds(kk * LANES, LANES)] = jnp.ones((LANES,), jnp.float32)
        pltpu.sync_copy(zbuf, hist.at[pl.ds(base, slab)])
        plsc.subcore_barrier()
        pltpu.sync_copy(dst_hbm.at[pl.ds(wid * kc, kc)], idx_v)

        def body(j, carry):
            pltpu.sync_copy(ones_v, hist.at[idx_v.at[j]], add=True)
            return carry

        lax.fori_loop(0, kc, body, 0)
        plsc.subcore_barrier()
        pltpu.sync_copy(hist.at[pl.ds(base, slab)], zbuf)
        pltpu.sync_copy(zbuf, out_hbm.at[pl.ds(c * np_ + base, slab)])

    return k(dstp)


@functools.partial(jax.jit, static_argnames=("np_", "kc0", "kc1", "d"))
def _scatter_call(hs, srcp, dstp, np_, kc0, kc1, d):
    slab = np_ // NS
    kcmax = max(kc0, kc1)
    mesh = plsc.VectorSubcoreMesh(**_MESH)

    @functools.partial(
        pl.kernel,
        out_type=jax.ShapeDtypeStruct((NC, np_, d), jnp.float32),
        mesh=mesh,
        scratch_types=[
            pltpu.VMEM((2, CHUNK), jnp.int32),
            pltpu.VMEM((2, CHUNK), jnp.int32),
            pltpu.VMEM((kcmax, CHUNK), jnp.int32),
            pltpu.VMEM((CHUNK, d), jnp.float32),
            pltpu.VMEM((CHUNK, d), jnp.float32),
            pltpu.VMEM_SHARED((np_, d), jnp.float32),
            pltpu.SemaphoreType.DMA,
            pltpu.SemaphoreType.DMA,
            pltpu.SemaphoreType.DMA,
            pltpu.SemaphoreType.DMA,
        ],
        compiler_params=pltpu.CompilerParams(use_tc_tiling_on_sc=False),
    )
    def k(hs_hbm, src_hbm, dst_hbm, out_hbm, srca, srcb, didx, buf, buf1,
          acc, sem, sem1, isema, isemb):
        c = lax.axis_index("c")
        s = lax.axis_index("s")
        base = s * slab

        def zb(rr, carry):
            for kk in range(d // LANES):
                buf[rr, pl.ds(kk * LANES, LANES)] = jnp.zeros((LANES,),
                                                              jnp.float32)
            return carry

        lax.fori_loop(0, CHUNK, zb, 0)
        nfull, rem = slab // CHUNK, slab % CHUNK
        for kk in range(nfull):
            pltpu.sync_copy(buf, acc.at[pl.ds(base + kk * CHUNK, CHUNK), :])
        if rem:
            pltpu.sync_copy(buf.at[pl.ds(0, rem), :],
                            acc.at[pl.ds(base + nfull * CHUNK, rem), :])
        plsc.subcore_barrier()

        dummy = hs_hbm.at[pl.ds(0, CHUNK), :]
        idummy = src_hbm.at[pl.ds(0, 2)]

        def half(j0, cur, nxt, isem_n, ioff):
            pltpu.async_copy(src_hbm.at[pl.ds(ioff, 2)], nxt, isem_n)
            pltpu.make_async_copy(dummy, buf, sem).wait()
            pltpu.async_copy(hs_hbm.at[cur.at[1]], buf1, sem1)
            pltpu.sync_copy(buf, acc.at[didx.at[j0]], add=True)
            pltpu.make_async_copy(dummy, buf1, sem1).wait()
            pltpu.make_async_copy(idummy, nxt, isem_n).wait()
            pltpu.async_copy(hs_hbm.at[nxt.at[0]], buf, sem)
            pltpu.sync_copy(buf1, acc.at[didx.at[j0 + 1]], add=True)

        def mainloop(cbase, kcc):
            pltpu.sync_copy(dst_hbm.at[pl.ds(cbase, kcc)],
                            didx.at[pl.ds(0, kcc)])
            pltpu.sync_copy(src_hbm.at[pl.ds(cbase, 2)], srca)
            pltpu.async_copy(hs_hbm.at[srca.at[0]], buf, sem)

            def body(m, carry):
                j0 = m * 4
                half(j0, srca, srcb, isemb, cbase + j0 + 2)
                half(j0 + 2, srcb, srca, isema,
                     cbase + jnp.minimum(j0 + 4, kcc - 2))
                return carry

            lax.fori_loop(0, kcc // 4, body, 0)
            pltpu.make_async_copy(dummy, buf, sem).wait()

        @pl.when(c == 0)
        def _():
            mainloop(s * kc0, kc0)

        @pl.when(c == 1)
        def _():
            mainloop(NS * kc0 + s * kc1, kc1)

        plsc.subcore_barrier()
        for kk in range(nfull):
            rows = pl.ds(base + kk * CHUNK, CHUNK)
            pltpu.sync_copy(acc.at[rows, :], buf)
            pltpu.sync_copy(buf, out_hbm.at[c, rows, :])
        if rem:
            rows = pl.ds(base + nfull * CHUNK, rem)
            pltpu.sync_copy(acc.at[rows, :], buf.at[pl.ds(0, rem), :])
            pltpu.sync_copy(buf.at[pl.ds(0, rem), :], out_hbm.at[c, rows, :])

    return k(hs, srcp, dstp)


def _c0_body(deg_ref, x_ref, w_ref, hs_ref, dinv_ref):
    d = deg_ref[...]
    dv = lax.rsqrt(d[:, 0:1] + d[:, 1:2] + 1.0)
    h = jnp.dot(x_ref[...], w_ref[...],
                preferred_element_type=jnp.float32,
                precision=lax.Precision.DEFAULT)
    hs_ref[...] = h * dv
    dinv_ref[...] = dv


@functools.partial(jax.jit, static_argnames=("np_", "r"))
def _c0_call(degt, xp, w1, np_, r):
    nb = np_ // r
    din = xp.shape[1]
    return pl.pallas_call(
        _c0_body,
        grid=(nb,),
        in_specs=[
            pl.BlockSpec((r, 2), lambda i: (i, 0)),
            pl.BlockSpec((r, din), lambda i: (i, 0)),
            pl.BlockSpec((din, din), lambda i: (0, 0)),
        ],
        out_specs=[
            pl.BlockSpec((r, din), lambda i: (i, 0)),
            pl.BlockSpec((r, 1), lambda i: (i, 0)),
        ],
        out_shape=[
            jax.ShapeDtypeStruct((np_, din), jnp.float32),
            jax.ShapeDtypeStruct((np_, 1), jnp.float32),
        ],
    )(degt, xp, w1)


def _ca_body(n, r, acc_ref, hs_ref, dinv_ref, b_ref, z_ref, s1_ref, s2_ref):
    i = pl.program_id(0)
    a = acc_ref[0] + acc_ref[1]
    pre = dinv_ref[...] * (a + hs_ref[...]) + b_ref[...]
    z = jnp.maximum(pre, 0.0)
    rowid = lax.broadcasted_iota(jnp.int32, (r, 1), 0) + i * r
    z = jnp.where(rowid < n, z, 0.0)
    z_ref[...] = z

    @pl.when(i == 0)
    def _():
        s1_ref[...] = jnp.zeros_like(s1_ref)
        s2_ref[...] = jnp.zeros_like(s2_ref)

    s1_ref[...] += jnp.sum(z, axis=0, keepdims=True)
    s2_ref[...] += jnp.sum(z * z, axis=0, keepdims=True)


@functools.partial(jax.jit, static_argnames=("n", "np_", "r"))
def _ca_call(acc, hs, dinv, b, n, np_, r):
    nb = np_ // r
    d = hs.shape[1]
    return pl.pallas_call(
        functools.partial(_ca_body, n, r),
        grid=(nb,),
        in_specs=[
            pl.BlockSpec((NC, r, d), lambda i: (0, i, 0)),
            pl.BlockSpec((r, d), lambda i: (i, 0)),
            pl.BlockSpec((r, 1), lambda i: (i, 0)),
            pl.BlockSpec((1, d), lambda i: (0, 0)),
        ],
        out_specs=[
            pl.BlockSpec((r, d), lambda i: (i, 0)),
            pl.BlockSpec((1, d), lambda i: (0, 0)),
            pl.BlockSpec((1, d), lambda i: (0, 0)),
        ],
        out_shape=[
            jax.ShapeDtypeStruct((np_, d), jnp.float32),
            jax.ShapeDtypeStruct((1, d), jnp.float32),
            jax.ShapeDtypeStruct((1, d), jnp.float32),
        ],
    )(acc, hs, dinv, b)


def _cb_body(n, z_ref, s1_ref, s2_ref, g_ref, be_ref, w_ref, dinv_ref,
             hs_ref):
    m = s1_ref[...] * (1.0 / n)
    v = s2_ref[...] * (1.0 / n) - m * m
    sc = g_ref[...] * lax.rsqrt(v + 1e-5)
    y = (z_ref[...] - m) * sc + be_ref[...]
    h = jnp.dot(y, w_ref[...],
                preferred_element_type=jnp.float32,
                precision=lax.Precision.DEFAULT)
    hs_ref[...] = h * dinv_ref[...]


@functools.partial(jax.jit, static_argnames=("n", "np_", "r"))
def _cb_call(z, s1, s2, g, be, w, dinv, n, np_, r):
    nb = np_ // r
    d = z.shape[1]
    dout = w.shape[1]
    return pl.pallas_call(
        functools.partial(_cb_body, n),
        grid=(nb,),
        in_specs=[
            pl.BlockSpec((r, d), lambda i: (i, 0)),
            pl.BlockSpec((1, d), lambda i: (0, 0)),
            pl.BlockSpec((1, d), lambda i: (0, 0)),
            pl.BlockSpec((1, d), lambda i: (0, 0)),
            pl.BlockSpec((1, d), lambda i: (0, 0)),
            pl.BlockSpec((d, dout), lambda i: (0, 0)),
            pl.BlockSpec((r, 1), lambda i: (i, 0)),
        ],
        out_specs=pl.BlockSpec((r, dout), lambda i: (i, 0)),
        out_shape=jax.ShapeDtypeStruct((np_, dout), jnp.float32),
    )(z, s1, s2, g, be, w, dinv)


def _cf_body(n, z_ref, s1_ref, s2_ref, g_ref, be_ref, out_ref):
    m = s1_ref[...] * (1.0 / n)
    v = s2_ref[...] * (1.0 / n) - m * m
    sc = g_ref[...] * lax.rsqrt(v + 1e-5)
    out_ref[...] = (z_ref[...] - m) * sc + be_ref[...]


@functools.partial(jax.jit, static_argnames=("n", "r"))
def _cf_call(z, s1, s2, g, be, n, r):
    nb = n // r
    d = z.shape[1]
    return pl.pallas_call(
        functools.partial(_cf_body, n),
        grid=(nb,),
        in_specs=[
            pl.BlockSpec((r, d), lambda i: (i, 0)),
            pl.BlockSpec((1, d), lambda i: (0, 0)),
            pl.BlockSpec((1, d), lambda i: (0, 0)),
            pl.BlockSpec((1, d), lambda i: (0, 0)),
            pl.BlockSpec((1, d), lambda i: (0, 0)),
        ],
        out_specs=pl.BlockSpec((r, d), lambda i: (i, 0)),
        out_shape=jax.ShapeDtypeStruct((n, d), jnp.float32),
    )(z, s1, s2, g, be)


def kernel(x, edge_index, W1, b1, g1, be1, W2, b2, g2, be2, W3, b3, g3, be3,
           W4, b4, g4, be4):
    n, din = x.shape
    e = edge_index.shape[1]
    np_ = ((n + 1 + 127) // 128) * 128
    kc = -(-e // (NC * NS * CHUNK))
    kc = ((kc + 7) // 8) * 8
    kct = 2 * kc
    kc0 = max(4, int(round(kct * _CORE0_FRAC / 4)) * 4)
    kc1 = kct - kc0
    epad = kc * NC * NS * CHUNK
    r = np_ // 8

    pad_idx = jnp.full((epad - e,), n, jnp.int32)
    srcp = jnp.concatenate([edge_index[0], pad_idx]).reshape(-1, CHUNK)
    dstp = jnp.concatenate([edge_index[1], pad_idx]).reshape(-1, CHUNK)
    xp = jnp.pad(x, ((0, np_ - n), (0, 0)))

    deg = _deg_call(dstp, np_=np_, kc=kc).reshape(NC, np_)
    degt = deg.T

    hs, dinv = _c0_call(degt, xp, W1, np_=np_, r=r)
    layers = [(b1, g1, be1, W2), (b2, g2, be2, W3), (b3, g3, be3, W4)]
    for b, g, be, wnext in layers:
        d = hs.shape[1]
        acc = _scatter_call(hs, srcp, dstp, np_=np_, kc0=kc0, kc1=kc1, d=d)
        z, s1, s2 = _ca_call(acc, hs, dinv, b.reshape(1, d), n=n, np_=np_,
                             r=r)
        hs = _cb_call(z, s1, s2, g.reshape(1, d), be.reshape(1, d), wnext,
                      dinv, n=n, np_=np_, r=r)
    d = hs.shape[1]
    acc = _scatter_call(hs, srcp, dstp, np_=np_, kc0=kc0, kc1=kc1, d=d)
    z, s1, s2 = _ca_call(acc, hs, dinv, b4.reshape(1, d), n=n, np_=np_, r=r)
    return _cf_call(z, s1, s2, g4.reshape(1, d), be4.reshape(1, d), n=n,
                    r=n // 10)


# speedup vs baseline: 1.1790x; 1.1790x over previous
import functools

import jax
import jax.numpy as jnp
from jax import lax
from jax.experimental import pallas as pl
from jax.experimental.pallas import tpu as pltpu
from jax.experimental.pallas import tpu_sc as plsc

NC = 2
NS = 16
LANES = 16
CHUNK = 128
_CORE0_FRAC = 0.75

_MESH = dict(core_axis_name="c", subcore_axis_name="s", num_cores=NC,
             num_subcores=NS)


def _vzero(ref, n):
    for k in range(n // LANES):
        ref[pl.ds(k * LANES, LANES)] = jnp.zeros((LANES,), jnp.float32)
    if n % LANES:
        ref[pl.ds(n - LANES, LANES)] = jnp.zeros((LANES,), jnp.float32)


@functools.partial(jax.jit, static_argnames=("np_", "kc"))
def _deg_call(dstp, np_, kc):
    slab = np_ // NS
    mesh = plsc.VectorSubcoreMesh(**_MESH)

    @functools.partial(
        pl.kernel,
        out_type=jax.ShapeDtypeStruct((NC * np_,), jnp.float32),
        mesh=mesh,
        scratch_types=[
            pltpu.VMEM((kc, CHUNK), jnp.int32),
            pltpu.VMEM((CHUNK,), jnp.float32),
            pltpu.VMEM((slab,), jnp.float32),
            pltpu.VMEM_SHARED((np_,), jnp.float32),
            pltpu.SemaphoreType.DMA,
        ],
    )
    def k(dst_hbm, out_hbm, idx_v, ones_v, zbuf, hist, sem):
        c = lax.axis_index("c")
        s = lax.axis_index("s")
        wid = s * NC + c
        base = s * slab
        _vzero(zbuf, slab)
        for kk in range(CHUNK // LANES):
            ones_v[pl.ds(kk * LANES, LANES)] = jnp.ones((LANES,), jnp.float32)
        pltpu.sync_copy(zbuf, hist.at[pl.ds(base, slab)])
        plsc.subcore_barrier()
        pltpu.sync_copy(dst_hbm.at[pl.ds(wid * kc, kc)], idx_v)

        def body(j, carry):
            pltpu.sync_copy(ones_v, hist.at[idx_v.at[j]], add=True)
            return carry

        lax.fori_loop(0, kc, body, 0)
        plsc.subcore_barrier()
        pltpu.sync_copy(hist.at[pl.ds(base, slab)], zbuf)
        pltpu.sync_copy(zbuf, out_hbm.at[pl.ds(c * np_ + base, slab)])

    return k(dstp)


@functools.partial(jax.jit, static_argnames=("np_", "kc0", "kc1", "d"))
def _scatter_call(hs, srcp, dstp, np_, kc0, kc1, d):
    slab = np_ // NS
    kcmax = max(kc0, kc1)
    mesh = plsc.VectorSubcoreMesh(**_MESH)

    @functools.partial(
        pl.kernel,
        out_type=jax.ShapeDtypeStruct((NC, np_, d), jnp.float32),
        mesh=mesh,
        scratch_types=[
            pltpu.VMEM((2, CHUNK), jnp.int32),
            pltpu.VMEM((2, CHUNK), jnp.int32),
            pltpu.VMEM((kcmax, CHUNK), jnp.int32),
            pltpu.VMEM((CHUNK, d), jnp.float32),
            pltpu.VMEM((CHUNK, d), jnp.float32),
            pltpu.VMEM_SHARED((np_, d), jnp.float32),
            pltpu.SemaphoreType.DMA,
            pltpu.SemaphoreType.DMA,
            pltpu.SemaphoreType.DMA,
            pltpu.SemaphoreType.DMA,
        ],
        compiler_params=pltpu.CompilerParams(use_tc_tiling_on_sc=False),
    )
    def k(hs_hbm, src_hbm, dst_hbm, out_hbm, srca, srcb, didx, buf, buf1,
          acc, sem, sem1, isema, isemb):
        c = lax.axis_index("c")
        s = lax.axis_index("s")
        base = s * slab

        def zb(rr, carry):
            for kk in range(d // LANES):
                buf[rr, pl.ds(kk * LANES, LANES)] = jnp.zeros((LANES,),
                                                              jnp.float32)
            return carry

        lax.fori_loop(0, CHUNK, zb, 0)
        nfull, rem = slab // CHUNK, slab % CHUNK
        for kk in range(nfull):
            pltpu.sync_copy(buf, acc.at[pl.ds(base + kk * CHUNK, CHUNK), :])
        if rem:
            pltpu.sync_copy(buf.at[pl.ds(0, rem), :],
                            acc.at[pl.ds(base + nfull * CHUNK, rem), :])
        plsc.subcore_barrier()

        dummy = hs_hbm.at[pl.ds(0, CHUNK), :]
        idummy = src_hbm.at[pl.ds(0, 2)]

        def half(j0, cur, nxt, isem_n, ioff):
            pltpu.async_copy(src_hbm.at[pl.ds(ioff, 2)], nxt, isem_n)
            pltpu.make_async_copy(dummy, buf, sem).wait()
            pltpu.async_copy(hs_hbm.at[cur.at[1]], buf1, sem1)
            pltpu.sync_copy(buf, acc.at[didx.at[j0]], add=True)
            pltpu.make_async_copy(dummy, buf1, sem1).wait()
            pltpu.make_async_copy(idummy, nxt, isem_n).wait()
            pltpu.async_copy(hs_hbm.at[nxt.at[0]], buf, sem)
            pltpu.sync_copy(buf1, acc.at[didx.at[j0 + 1]], add=True)

        def mainloop(cbase, kcc):
            pltpu.sync_copy(dst_hbm.at[pl.ds(cbase, kcc)],
                            didx.at[pl.ds(0, kcc)])
            pltpu.sync_copy(src_hbm.at[pl.ds(cbase, 2)], srca)
            pltpu.async_copy(hs_hbm.at[srca.at[0]], buf, sem)

            def body(m, carry):
                j0 = m * 4
                half(j0, srca, srcb, isemb, cbase + j0 + 2)
                half(j0 + 2, srcb, srca, isema,
                     cbase + jnp.minimum(j0 + 4, kcc - 2))
                return carry

            lax.fori_loop(0, kcc // 4, body, 0)
            pltpu.make_async_copy(dummy, buf, sem).wait()

        @pl.when(c == 0)
        def _():
            mainloop(s * kc0, kc0)

        @pl.when(c == 1)
        def _():
            mainloop(NS * kc0 + s * kc1, kc1)

        plsc.subcore_barrier()
        for kk in range(nfull):
            rows = pl.ds(base + kk * CHUNK, CHUNK)
            pltpu.sync_copy(acc.at[rows, :], buf)
            pltpu.sync_copy(buf, out_hbm.at[c, rows, :])
        if rem:
            rows = pl.ds(base + nfull * CHUNK, rem)
            pltpu.sync_copy(acc.at[rows, :], buf.at[pl.ds(0, rem), :])
            pltpu.sync_copy(buf.at[pl.ds(0, rem), :], out_hbm.at[c, rows, :])

    return k(hs, srcp, dstp)


def _c0_body(deg_ref, x_ref, w_ref, hs_ref, dinv_ref):
    d = deg_ref[...]
    dv = lax.rsqrt(d[:, 0:1] + d[:, 1:2] + 1.0)
    h = jnp.dot(x_ref[...], w_ref[...],
                preferred_element_type=jnp.float32,
                precision=lax.Precision.DEFAULT)
    hs_ref[...] = h * dv
    dinv_ref[...] = dv


@functools.partial(jax.jit, static_argnames=("np_", "r"))
def _c0_call(degt, xp, w1, np_, r):
    nb = np_ // r
    din = xp.shape[1]
    return pl.pallas_call(
        _c0_body,
        grid=(nb,),
        in_specs=[
            pl.BlockSpec((r, 2), lambda i: (i, 0)),
            pl.BlockSpec((r, din), lambda i: (i, 0)),
            pl.BlockSpec((din, din), lambda i: (0, 0)),
        ],
        out_specs=[
            pl.BlockSpec((r, din), lambda i: (i, 0)),
            pl.BlockSpec((r, 1), lambda i: (i, 0)),
        ],
        out_shape=[
            jax.ShapeDtypeStruct((np_, din), jnp.float32),
            jax.ShapeDtypeStruct((np_, 1), jnp.float32),
        ],
    )(degt, xp, w1)


def _ca_body(n, r, acc_ref, hs_ref, dinv_ref, b_ref, z_ref, s1_ref, s2_ref):
    i = pl.program_id(0)
    a = acc_ref[0] + acc_ref[1]
    pre = dinv_ref[...] * (a + hs_ref[...]) + b_ref[...]
    z = jnp.maximum(pre, 0.0)
    rowid = lax.broadcasted_iota(jnp.int32, (r, 1), 0) + i * r
    z = jnp.where(rowid < n, z, 0.0)
    z_ref[...] = z

    @pl.when(i == 0)
    def _():
        s1_ref[...] = jnp.zeros_like(s1_ref)
        s2_ref[...] = jnp.zeros_like(s2_ref)

    s1_ref[...] += jnp.sum(z, axis=0, keepdims=True)
    s2_ref[...] += jnp.sum(z * z, axis=0, keepdims=True)


@functools.partial(jax.jit, static_argnames=("n", "np_", "r"))
def _ca_call(acc, hs, dinv, b, n, np_, r):
    nb = np_ // r
    d = hs.shape[1]
    return pl.pallas_call(
        functools.partial(_ca_body, n, r),
        grid=(nb,),
        in_specs=[
            pl.BlockSpec((NC, r, d), lambda i: (0, i, 0)),
            pl.BlockSpec((r, d), lambda i: (i, 0)),
            pl.BlockSpec((r, 1), lambda i: (i, 0)),
            pl.BlockSpec((1, d), lambda i: (0, 0)),
        ],
        out_specs=[
            pl.BlockSpec((r, d), lambda i: (i, 0)),
            pl.BlockSpec((1, d), lambda i: (0, 0)),
            pl.BlockSpec((1, d), lambda i: (0, 0)),
        ],
        out_shape=[
            jax.ShapeDtypeStruct((np_, d), jnp.float32),
            jax.ShapeDtypeStruct((1, d), jnp.float32),
            jax.ShapeDtypeStruct((1, d), jnp.float32),
        ],
    )(acc, hs, dinv, b)


def _cb_body(n, z_ref, s1_ref, s2_ref, g_ref, be_ref, w_ref, dinv_ref,
             hs_ref):
    m = s1_ref[...] * (1.0 / n)
    v = s2_ref[...] * (1.0 / n) - m * m
    sc = g_ref[...] * lax.rsqrt(v + 1e-5)
    y = (z_ref[...] - m) * sc + be_ref[...]
    h = jnp.dot(y, w_ref[...],
                preferred_element_type=jnp.float32,
                precision=lax.Precision.DEFAULT)
    hs_ref[...] = h * dinv_ref[...]


@functools.partial(jax.jit, static_argnames=("n", "np_", "r"))
def _cb_call(z, s1, s2, g, be, w, dinv, n, np_, r):
    nb = np_ // r
    d = z.shape[1]
    dout = w.shape[1]
    return pl.pallas_call(
        functools.partial(_cb_body, n),
        grid=(nb,),
        in_specs=[
            pl.BlockSpec((r, d), lambda i: (i, 0)),
            pl.BlockSpec((1, d), lambda i: (0, 0)),
            pl.BlockSpec((1, d), lambda i: (0, 0)),
            pl.BlockSpec((1, d), lambda i: (0, 0)),
            pl.BlockSpec((1, d), lambda i: (0, 0)),
            pl.BlockSpec((d, dout), lambda i: (0, 0)),
            pl.BlockSpec((r, 1), lambda i: (i, 0)),
        ],
        out_specs=pl.BlockSpec((r, dout), lambda i: (i, 0)),
        out_shape=jax.ShapeDtypeStruct((np_, dout), jnp.float32),
    )(z, s1, s2, g, be, w, dinv)


def _cf_body(n, z_ref, s1_ref, s2_ref, g_ref, be_ref, out_ref):
    m = s1_ref[...] * (1.0 / n)
    v = s2_ref[...] * (1.0 / n) - m * m
    sc = g_ref[...] * lax.rsqrt(v + 1e-5)
    out_ref[...] = (z_ref[...] - m) * sc + be_ref[...]


@functools.partial(jax.jit, static_argnames=("n", "r"))
def _cf_call(z, s1, s2, g, be, n, r):
    nb = n // r
    d = z.shape[1]
    return pl.pallas_call(
        functools.partial(_cf_body, n),
        grid=(nb,),
        in_specs=[
            pl.BlockSpec((r, d), lambda i: (i, 0)),
            pl.BlockSpec((1, d), lambda i: (0, 0)),
            pl.BlockSpec((1, d), lambda i: (0, 0)),
            pl.BlockSpec((1, d), lambda i: (0, 0)),
            pl.BlockSpec((1, d), lambda i: (0, 0)),
        ],
        out_specs=pl.BlockSpec((r, d), lambda i: (i, 0)),
        out_shape=jax.ShapeDtypeStruct((n, d), jnp.float32),
    )(z, s1, s2, g, be)


def kernel(x, edge_index, W1, b1, g1, be1, W2, b2, g2, be2, W3, b3, g3, be3,
           W4, b4, g4, be4):
    n, din = x.shape
    e = edge_index.shape[1]
    np_ = ((n + 1 + 127) // 128) * 128
    kc = -(-e // (NC * NS * CHUNK))
    kc = ((kc + 7) // 8) * 8
    kct = 2 * kc
    kc0 = max(4, int(round(kct * _CORE0_FRAC / 4)) * 4)
    kc1 = kct - kc0
    epad = kc * NC * NS * CHUNK
    r = np_ // 8

    pad_idx = jnp.full((epad - e,), n, jnp.int32)
    srcp = jnp.concatenate([edge_index[0], pad_idx]).reshape(-1, CHUNK)
    dstp = jnp.concatenate([edge_index[1], pad_idx]).reshape(-1, CHUNK)
    xp = jnp.pad(x, ((0, np_ - n), (0, 0)))

    deg = _deg_call(dstp, np_=np_, kc=kc).reshape(NC, np_)
    degt = deg.T

    hs, dinv = _c0_call(degt, xp, W1, np_=np_, r=r)
    layers = [(b1, g1, be1, W2), (b2, g2, be2, W3), (b3, g3, be3, W4)]
    for b, g, be, wnext in layers:
        d = hs.shape[1]
        acc = _scatter_call(hs, srcp, dstp, np_=np_, kc0=kc0, kc1=kc1, d=d)
        z, s1, s2 = _ca_call(acc, hs, dinv, b.reshape(1, d), n=n, np_=np_,
                             r=r)
        hs = _cb_call(z, s1, s2, g.reshape(1, d), be.reshape(1, d), wnext,
                      dinv, n=n, np_=np_, r=r)
    d = hs.shape[1]
    acc = _scatter_call(hs, srcp, dstp, np_=np_, kc0=kc0, kc1=kc1, d=d)
    z, s1, s2 = _ca_call(acc, hs, dinv, b4.reshape(1, d), n=n, np_=np_, r=r)
    return _cf_call(z, s1, s2, g4.reshape(1, d), be4.reshape(1, d), n=n,
                    r=n // 10)


# speedup vs baseline: 2.5851x; 2.1925x over previous
import functools

import jax
import jax.numpy as jnp
from jax import lax
from jax.experimental import pallas as pl
from jax.experimental.pallas import tpu as pltpu
from jax.experimental.pallas import tpu_sc as plsc

NC = 2
NS = 16
LANES = 16
CHUNK = 128

_MESH = dict(core_axis_name="c", subcore_axis_name="s", num_cores=NC,
             num_subcores=NS)


def _vzero(ref, n):
    for k in range(n // LANES):
        ref[pl.ds(k * LANES, LANES)] = jnp.zeros((LANES,), jnp.float32)
    if n % LANES:
        ref[pl.ds(n - LANES, LANES)] = jnp.zeros((LANES,), jnp.float32)


@functools.partial(jax.jit, static_argnames=("np_", "kc"))
def _deg_call(dstp, np_, kc):
    slab = np_ // NS
    mesh = plsc.VectorSubcoreMesh(**_MESH)

    @functools.partial(
        pl.kernel,
        out_type=jax.ShapeDtypeStruct((NC * np_,), jnp.float32),
        mesh=mesh,
        scratch_types=[
            pltpu.VMEM((kc, CHUNK), jnp.int32),
            pltpu.VMEM((CHUNK,), jnp.float32),
            pltpu.VMEM((slab,), jnp.float32),
            pltpu.VMEM_SHARED((np_,), jnp.float32),
            pltpu.SemaphoreType.DMA,
        ],
    )
    def k(dst_hbm, out_hbm, idx_v, ones_v, zbuf, hist, sem):
        c = lax.axis_index("c")
        s = lax.axis_index("s")
        wid = s * NC + c
        base = s * slab
        _vzero(zbuf, slab)
        for kk in range(CHUNK // LANES):
            ones_v[pl.ds(kk * LANES, LANES)] = jnp.ones((LANES,), jnp.float32)
        pltpu.sync_copy(zbuf, hist.at[pl.ds(base, slab)])
        plsc.subcore_barrier()
        pltpu.sync_copy(dst_hbm.at[pl.ds(wid * kc, kc)], idx_v)

        def body(j, carry):
            pltpu.sync_copy(ones_v, hist.at[idx_v.at[j]], add=True)
            return carry

        lax.fori_loop(0, kc, body, 0)
        plsc.subcore_barrier()
        pltpu.sync_copy(hist.at[pl.ds(base, slab)], zbuf)
        pltpu.sync_copy(zbuf, out_hbm.at[pl.ds(c * np_ + base, slab)])

    return k(dstp)


@functools.partial(jax.jit, static_argnames=("np_", "kt", "d"))
def _scatter_call(hs, srcp, dstp, np_, kt, d):
    slab = np_ // NS
    dh = d // 2
    mesh = plsc.VectorSubcoreMesh(**_MESH)

    @functools.partial(
        pl.kernel,
        out_type=jax.ShapeDtypeStruct((np_, d), jnp.float32),
        mesh=mesh,
        scratch_types=[
            pltpu.VMEM((2, CHUNK), jnp.int32),
            pltpu.VMEM((2, CHUNK), jnp.int32),
            pltpu.VMEM((kt, CHUNK), jnp.int32),
            pltpu.VMEM((CHUNK, dh), jnp.float32),
            pltpu.VMEM((CHUNK, dh), jnp.float32),
            pltpu.VMEM_SHARED((np_, dh), jnp.float32),
            pltpu.VMEM_SHARED((np_, dh), jnp.float32),
            pltpu.SemaphoreType.DMA,
            pltpu.SemaphoreType.DMA,
            pltpu.SemaphoreType.DMA,
            pltpu.SemaphoreType.DMA,
        ],
        compiler_params=pltpu.CompilerParams(use_tc_tiling_on_sc=False),
    )
    def k(hs_hbm, src_hbm, dst_hbm, out_hbm, srca, srcb, didx, buf, buf1,
          hsh, acc, sem, sem1, isema, isemb):
        c = lax.axis_index("c")
        s = lax.axis_index("s")
        base = s * slab
        cols = pl.ds(c * dh, dh)

        def zb(rr, carry):
            for kk in range(dh // LANES):
                buf[rr, pl.ds(kk * LANES, LANES)] = jnp.zeros((LANES,),
                                                              jnp.float32)
            return carry

        lax.fori_loop(0, CHUNK, zb, 0)
        nfull, rem = slab // CHUNK, slab % CHUNK
        for kk in range(nfull):
            pltpu.sync_copy(buf, acc.at[pl.ds(base + kk * CHUNK, CHUNK), :])
        if rem:
            pltpu.sync_copy(buf.at[pl.ds(0, rem), :],
                            acc.at[pl.ds(base + nfull * CHUNK, rem), :])
        for kk in range(nfull + (1 if rem else 0)):
            nr = CHUNK if kk < nfull else rem
            rows = pl.ds(base + kk * CHUNK, nr)
            pltpu.sync_copy(hs_hbm.at[rows, cols], buf1.at[pl.ds(0, nr), :])
            pltpu.sync_copy(buf1.at[pl.ds(0, nr), :], hsh.at[rows, :])
        plsc.subcore_barrier()

        dummy = hs_hbm.at[pl.ds(0, CHUNK), cols]
        idummy = src_hbm.at[pl.ds(0, 2)]
        cbase = s * kt
        pltpu.sync_copy(dst_hbm.at[pl.ds(cbase, kt)], didx)
        pltpu.sync_copy(src_hbm.at[pl.ds(cbase, 2)], srca)
        pltpu.async_copy(hsh.at[srca.at[0]], buf, sem)

        def half(j0, cur, nxt, isem_n, ioff):
            pltpu.async_copy(src_hbm.at[pl.ds(ioff, 2)], nxt, isem_n)
            pltpu.make_async_copy(dummy, buf, sem).wait()
            pltpu.async_copy(hsh.at[cur.at[1]], buf1, sem1)
            pltpu.sync_copy(buf, acc.at[didx.at[j0]], add=True)
            pltpu.make_async_copy(dummy, buf1, sem1).wait()
            pltpu.make_async_copy(idummy, nxt, isem_n).wait()
            pltpu.async_copy(hsh.at[nxt.at[0]], buf, sem)
            pltpu.sync_copy(buf1, acc.at[didx.at[j0 + 1]], add=True)

        def body(m, carry):
            j0 = m * 4
            half(j0, srca, srcb, isemb, cbase + j0 + 2)
            half(j0 + 2, srcb, srca, isema,
                 cbase + jnp.minimum(j0 + 4, kt - 2))
            return carry

        lax.fori_loop(0, kt // 4, body, 0)
        pltpu.make_async_copy(dummy, buf, sem).wait()
        plsc.subcore_barrier()
        for kk in range(nfull):
            rows = pl.ds(base + kk * CHUNK, CHUNK)
            pltpu.sync_copy(acc.at[rows, :], buf)
            pltpu.sync_copy(buf, out_hbm.at[rows, cols])
        if rem:
            rows = pl.ds(base + nfull * CHUNK, rem)
            pltpu.sync_copy(acc.at[rows, :], buf.at[pl.ds(0, rem), :])
            pltpu.sync_copy(buf.at[pl.ds(0, rem), :], out_hbm.at[rows, cols])

    return k(hs, srcp, dstp)


def _c0_body(deg_ref, x_ref, w_ref, hs_ref, dinv_ref):
    d = deg_ref[...]
    dv = lax.rsqrt(d[:, 0:1] + d[:, 1:2] + 1.0)
    h = jnp.dot(x_ref[...], w_ref[...],
                preferred_element_type=jnp.float32,
                precision=lax.Precision.DEFAULT)
    hs_ref[...] = h * dv
    dinv_ref[...] = dv


@functools.partial(jax.jit, static_argnames=("np_", "r"))
def _c0_call(degt, xp, w1, np_, r):
    nb = np_ // r
    din = xp.shape[1]
    return pl.pallas_call(
        _c0_body,
        grid=(nb,),
        in_specs=[
            pl.BlockSpec((r, 2), lambda i: (i, 0)),
            pl.BlockSpec((r, din), lambda i: (i, 0)),
            pl.BlockSpec((din, din), lambda i: (0, 0)),
        ],
        out_specs=[
            pl.BlockSpec((r, din), lambda i: (i, 0)),
            pl.BlockSpec((r, 1), lambda i: (i, 0)),
        ],
        out_shape=[
            jax.ShapeDtypeStruct((np_, din), jnp.float32),
            jax.ShapeDtypeStruct((np_, 1), jnp.float32),
        ],
    )(degt, xp, w1)


def _ca_body(n, r, acc_ref, hs_ref, dinv_ref, b_ref, z_ref, s1_ref, s2_ref):
    i = pl.program_id(0)
    pre = dinv_ref[...] * (acc_ref[...] + hs_ref[...]) + b_ref[...]
    z = jnp.maximum(pre, 0.0)
    rowid = lax.broadcasted_iota(jnp.int32, (r, 1), 0) + i * r
    z = jnp.where(rowid < n, z, 0.0)
    z_ref[...] = z

    @pl.when(i == 0)
    def _():
        s1_ref[...] = jnp.zeros_like(s1_ref)
        s2_ref[...] = jnp.zeros_like(s2_ref)

    s1_ref[...] += jnp.sum(z, axis=0, keepdims=True)
    s2_ref[...] += jnp.sum(z * z, axis=0, keepdims=True)


@functools.partial(jax.jit, static_argnames=("n", "np_", "r"))
def _ca_call(acc, hs, dinv, b, n, np_, r):
    nb = np_ // r
    d = hs.shape[1]
    return pl.pallas_call(
        functools.partial(_ca_body, n, r),
        grid=(nb,),
        in_specs=[
            pl.BlockSpec((r, d), lambda i: (i, 0)),
            pl.BlockSpec((r, d), lambda i: (i, 0)),
            pl.BlockSpec((r, 1), lambda i: (i, 0)),
            pl.BlockSpec((1, d), lambda i: (0, 0)),
        ],
        out_specs=[
            pl.BlockSpec((r, d), lambda i: (i, 0)),
            pl.BlockSpec((1, d), lambda i: (0, 0)),
            pl.BlockSpec((1, d), lambda i: (0, 0)),
        ],
        out_shape=[
            jax.ShapeDtypeStruct((np_, d), jnp.float32),
            jax.ShapeDtypeStruct((1, d), jnp.float32),
            jax.ShapeDtypeStruct((1, d), jnp.float32),
        ],
    )(acc, hs, dinv, b)


def _cb_body(n, z_ref, s1_ref, s2_ref, g_ref, be_ref, w_ref, dinv_ref,
             hs_ref):
    m = s1_ref[...] * (1.0 / n)
    v = s2_ref[...] * (1.0 / n) - m * m
    sc = g_ref[...] * lax.rsqrt(v + 1e-5)
    y = (z_ref[...] - m) * sc + be_ref[...]
    h = jnp.dot(y, w_ref[...],
                preferred_element_type=jnp.float32,
                precision=lax.Precision.DEFAULT)
    hs_ref[...] = h * dinv_ref[...]


@functools.partial(jax.jit, static_argnames=("n", "np_", "r"))
def _cb_call(z, s1, s2, g, be, w, dinv, n, np_, r):
    nb = np_ // r
    d = z.shape[1]
    dout = w.shape[1]
    return pl.pallas_call(
        functools.partial(_cb_body, n),
        grid=(nb,),
        in_specs=[
            pl.BlockSpec((r, d), lambda i: (i, 0)),
            pl.BlockSpec((1, d), lambda i: (0, 0)),
            pl.BlockSpec((1, d), lambda i: (0, 0)),
            pl.BlockSpec((1, d), lambda i: (0, 0)),
            pl.BlockSpec((1, d), lambda i: (0, 0)),
            pl.BlockSpec((d, dout), lambda i: (0, 0)),
            pl.BlockSpec((r, 1), lambda i: (i, 0)),
        ],
        out_specs=pl.BlockSpec((r, dout), lambda i: (i, 0)),
        out_shape=jax.ShapeDtypeStruct((np_, dout), jnp.float32),
    )(z, s1, s2, g, be, w, dinv)


def _cf_body(n, z_ref, s1_ref, s2_ref, g_ref, be_ref, out_ref):
    m = s1_ref[...] * (1.0 / n)
    v = s2_ref[...] * (1.0 / n) - m * m
    sc = g_ref[...] * lax.rsqrt(v + 1e-5)
    out_ref[...] = (z_ref[...] - m) * sc + be_ref[...]


@functools.partial(jax.jit, static_argnames=("n", "r"))
def _cf_call(z, s1, s2, g, be, n, r):
    nb = n // r
    d = z.shape[1]
    return pl.pallas_call(
        functools.partial(_cf_body, n),
        grid=(nb,),
        in_specs=[
            pl.BlockSpec((r, d), lambda i: (i, 0)),
            pl.BlockSpec((1, d), lambda i: (0, 0)),
            pl.BlockSpec((1, d), lambda i: (0, 0)),
            pl.BlockSpec((1, d), lambda i: (0, 0)),
            pl.BlockSpec((1, d), lambda i: (0, 0)),
        ],
        out_specs=pl.BlockSpec((r, d), lambda i: (i, 0)),
        out_shape=jax.ShapeDtypeStruct((n, d), jnp.float32),
    )(z, s1, s2, g, be)


def kernel(x, edge_index, W1, b1, g1, be1, W2, b2, g2, be2, W3, b3, g3, be3,
           W4, b4, g4, be4):
    n, din = x.shape
    e = edge_index.shape[1]
    np_ = ((n + 1 + 127) // 128) * 128
    kc = -(-e // (NC * NS * CHUNK))
    kc = ((kc + 7) // 8) * 8
    kt = 2 * kc
    epad = kc * NC * NS * CHUNK
    r = np_ // 8

    pad_idx = jnp.full((epad - e,), n, jnp.int32)
    srcp = jnp.concatenate([edge_index[0], pad_idx]).reshape(-1, CHUNK)
    dstp = jnp.concatenate([edge_index[1], pad_idx]).reshape(-1, CHUNK)
    xp = jnp.pad(x, ((0, np_ - n), (0, 0)))

    deg = _deg_call(dstp, np_=np_, kc=kc).reshape(NC, np_)
    degt = deg.T

    hs, dinv = _c0_call(degt, xp, W1, np_=np_, r=r)
    layers = [(b1, g1, be1, W2), (b2, g2, be2, W3), (b3, g3, be3, W4)]
    for b, g, be, wnext in layers:
        d = hs.shape[1]
        acc = _scatter_call(hs, srcp, dstp, np_=np_, kt=kt, d=d)
        z, s1, s2 = _ca_call(acc, hs, dinv, b.reshape(1, d), n=n, np_=np_,
                             r=r)
        hs = _cb_call(z, s1, s2, g.reshape(1, d), be.reshape(1, d), wnext,
                      dinv, n=n, np_=np_, r=r)
    d = hs.shape[1]
    acc = _scatter_call(hs, srcp, dstp, np_=np_, kt=kt, d=d)
    z, s1, s2 = _ca_call(acc, hs, dinv, b4.reshape(1, d), n=n, np_=np_, r=r)
    return _cf_call(z, s1, s2, g4.reshape(1, d), be4.reshape(1, d), n=n,
                    r=n // 10)


# speedup vs baseline: 2.6681x; 1.0321x over previous
import functools

import jax
import jax.numpy as jnp
from jax import lax
from jax.experimental import pallas as pl
from jax.experimental.pallas import tpu as pltpu
from jax.experimental.pallas import tpu_sc as plsc

NC = 2
NS = 16
LANES = 16
CHUNK = 128

_MESH = dict(core_axis_name="c", subcore_axis_name="s", num_cores=NC,
             num_subcores=NS)


def _vzero(ref, n):
    for k in range(n // LANES):
        ref[pl.ds(k * LANES, LANES)] = jnp.zeros((LANES,), jnp.float32)
    if n % LANES:
        ref[pl.ds(n - LANES, LANES)] = jnp.zeros((LANES,), jnp.float32)


@functools.partial(jax.jit, static_argnames=("np_", "kc"))
def _deg_call(dstp, np_, kc):
    slab = np_ // NS
    mesh = plsc.VectorSubcoreMesh(**_MESH)

    @functools.partial(
        pl.kernel,
        out_type=jax.ShapeDtypeStruct((NC * np_,), jnp.float32),
        mesh=mesh,
        scratch_types=[
            pltpu.VMEM((kc, CHUNK), jnp.int32),
            pltpu.VMEM((CHUNK,), jnp.float32),
            pltpu.VMEM((slab,), jnp.float32),
            pltpu.VMEM_SHARED((np_,), jnp.float32),
            pltpu.SemaphoreType.DMA,
        ],
    )
    def k(dst_hbm, out_hbm, idx_v, ones_v, zbuf, hist, sem):
        c = lax.axis_index("c")
        s = lax.axis_index("s")
        wid = s * NC + c
        base = s * slab
        _vzero(zbuf, slab)
        for kk in range(CHUNK // LANES):
            ones_v[pl.ds(kk * LANES, LANES)] = jnp.ones((LANES,), jnp.float32)
        pltpu.sync_copy(zbuf, hist.at[pl.ds(base, slab)])
        plsc.subcore_barrier()
        pltpu.sync_copy(dst_hbm.at[pl.ds(wid * kc, kc)], idx_v)

        def body(j, carry):
            pltpu.sync_copy(ones_v, hist.at[idx_v.at[j]], add=True)
            return carry

        lax.fori_loop(0, kc, body, 0)
        plsc.subcore_barrier()
        pltpu.sync_copy(hist.at[pl.ds(base, slab)], zbuf)
        pltpu.sync_copy(zbuf, out_hbm.at[pl.ds(c * np_ + base, slab)])

    return k(dstp)


@functools.partial(jax.jit, static_argnames=("np_", "kt", "d"))
def _scatter_call(hs, srcp, dstp, np_, kt, d):
    slab = np_ // NS
    dh = d // 2
    mesh = plsc.VectorSubcoreMesh(**_MESH)

    @functools.partial(
        pl.kernel,
        out_type=jax.ShapeDtypeStruct((np_, d), jnp.float32),
        mesh=mesh,
        scratch_types=[
            pltpu.VMEM((2, CHUNK), jnp.int32),
            pltpu.VMEM((2, CHUNK), jnp.int32),
            pltpu.VMEM((kt, CHUNK), jnp.int32),
            pltpu.VMEM((CHUNK, dh), jnp.float32),
            pltpu.VMEM((CHUNK, dh), jnp.float32),
            pltpu.VMEM((CHUNK, dh), jnp.float32),
            pltpu.VMEM_SHARED((np_, dh), jnp.float32),
            pltpu.VMEM_SHARED((np_, dh), jnp.float32),
            pltpu.SemaphoreType.DMA,
            pltpu.SemaphoreType.DMA,
            pltpu.SemaphoreType.DMA,
            pltpu.SemaphoreType.DMA,
            pltpu.SemaphoreType.DMA,
            pltpu.SemaphoreType.DMA,
            pltpu.SemaphoreType.DMA,
            pltpu.SemaphoreType.DMA,
            pltpu.SemaphoreType.DMA,
        ],
        compiler_params=pltpu.CompilerParams(use_tc_tiling_on_sc=False),
    )
    def k(hs_hbm, src_hbm, dst_hbm, out_hbm, srca, srcb, didx, buf, buf1,
          buf2, hsh, acc, sem, sem1, isema, isemb, semz, semh1, semh2,
          sems1, sems2):
        c = lax.axis_index("c")
        s = lax.axis_index("s")
        base = s * slab
        cols = pl.ds(c * dh, dh)
        nfull, rem = slab // CHUNK, slab % CHUNK
        nf5 = nfull + (1 if rem else 0)
        bufs = [buf1, buf2]
        semh = [semh1, semh2]
        sems = [sems1, sems2]

        def run_pairs(hop1, hop2):
            for kk in range(nf5):
                if kk >= 2:
                    pltpu.make_async_copy(*hop2[kk - 2]).wait()
                pltpu.async_copy(*hop1[kk])
                if kk >= 1:
                    pltpu.make_async_copy(*hop1[kk - 1]).wait()
                    pltpu.async_copy(*hop2[kk - 1])
            pltpu.make_async_copy(*hop1[nf5 - 1]).wait()
            pltpu.async_copy(*hop2[nf5 - 1])
            pltpu.make_async_copy(*hop2[nf5 - 2]).wait()
            pltpu.make_async_copy(*hop2[nf5 - 1]).wait()

        def zb(rr, carry):
            for kk in range(dh // LANES):
                buf[rr, pl.ds(kk * LANES, LANES)] = jnp.zeros((LANES,),
                                                              jnp.float32)
            return carry

        lax.fori_loop(0, CHUNK, zb, 0)
        zdst = []
        for kk in range(nf5):
            nr = CHUNK if kk < nfull else rem
            zdst.append((buf.at[pl.ds(0, nr), :],
                         acc.at[pl.ds(base + kk * CHUNK, nr), :]))
            pltpu.async_copy(zdst[-1][0], zdst[-1][1], semz)
        cbase = s * kt
        pltpu.async_copy(dst_hbm.at[pl.ds(cbase, kt)], didx, isema)
        stage1, stage2 = [], []
        for kk in range(nf5):
            nr = CHUNK if kk < nfull else rem
            rows = pl.ds(base + kk * CHUNK, nr)
            bx = bufs[kk % 2].at[pl.ds(0, nr), :]
            stage1.append((hs_hbm.at[rows, cols], bx, semh[kk % 2]))
            stage2.append((bx, hsh.at[rows, :], sems[kk % 2]))
        run_pairs(stage1, stage2)
        for z in zdst:
            pltpu.make_async_copy(z[0], z[1], semz).wait()
        pltpu.make_async_copy(src_hbm.at[pl.ds(0, kt)], didx, isema).wait()
        plsc.subcore_barrier()

        dummy = hs_hbm.at[pl.ds(0, CHUNK), cols]
        idummy = src_hbm.at[pl.ds(0, 2)]
        pltpu.sync_copy(src_hbm.at[pl.ds(cbase, 2)], srca)
        pltpu.async_copy(hsh.at[srca.at[0]], buf, sem)

        def half(j0, cur, nxt, isem_n, ioff):
            pltpu.async_copy(src_hbm.at[pl.ds(ioff, 2)], nxt, isem_n)
            pltpu.make_async_copy(dummy, buf, sem).wait()
            pltpu.async_copy(hsh.at[cur.at[1]], buf1, sem1)
            pltpu.sync_copy(buf, acc.at[didx.at[j0]], add=True)
            pltpu.make_async_copy(dummy, buf1, sem1).wait()
            pltpu.make_async_copy(idummy, nxt, isem_n).wait()
            pltpu.async_copy(hsh.at[nxt.at[0]], buf, sem)
            pltpu.sync_copy(buf1, acc.at[didx.at[j0 + 1]], add=True)

        def body(m, carry):
            j0 = m * 4
            half(j0, srca, srcb, isemb, cbase + j0 + 2)
            half(j0 + 2, srcb, srca, isema,
                 cbase + jnp.minimum(j0 + 4, kt - 2))
            return carry

        lax.fori_loop(0, kt // 4, body, 0)
        pltpu.make_async_copy(dummy, buf, sem).wait()
        plsc.subcore_barrier()
        out1, out2 = [], []
        for kk in range(nf5):
            nr = CHUNK if kk < nfull else rem
            rows = pl.ds(base + kk * CHUNK, nr)
            bx = bufs[kk % 2].at[pl.ds(0, nr), :]
            out1.append((acc.at[rows, :], bx, semh[kk % 2]))
            out2.append((bx, out_hbm.at[rows, cols], sems[kk % 2]))
        run_pairs(out1, out2)

    return k(hs, srcp, dstp)


def _c0_body(deg_ref, x_ref, w_ref, hs_ref, dinv_ref):
    d = deg_ref[...]
    dv = lax.rsqrt(d[:, 0:1] + d[:, 1:2] + 1.0)
    h = jnp.dot(x_ref[...], w_ref[...],
                preferred_element_type=jnp.float32,
                precision=lax.Precision.DEFAULT)
    hs_ref[...] = h * dv
    dinv_ref[...] = dv


@functools.partial(jax.jit, static_argnames=("np_", "r"))
def _c0_call(degt, xp, w1, np_, r):
    nb = np_ // r
    din = xp.shape[1]
    return pl.pallas_call(
        _c0_body,
        grid=(nb,),
        in_specs=[
            pl.BlockSpec((r, 2), lambda i: (i, 0)),
            pl.BlockSpec((r, din), lambda i: (i, 0)),
            pl.BlockSpec((din, din), lambda i: (0, 0)),
        ],
        out_specs=[
            pl.BlockSpec((r, din), lambda i: (i, 0)),
            pl.BlockSpec((r, 1), lambda i: (i, 0)),
        ],
        out_shape=[
            jax.ShapeDtypeStruct((np_, din), jnp.float32),
            jax.ShapeDtypeStruct((np_, 1), jnp.float32),
        ],
    )(degt, xp, w1)


def _ca_body(n, r, acc_ref, hs_ref, dinv_ref, b_ref, z_ref, s1_ref, s2_ref):
    i = pl.program_id(0)
    pre = dinv_ref[...] * (acc_ref[...] + hs_ref[...]) + b_ref[...]
    z = jnp.maximum(pre, 0.0)
    rowid = lax.broadcasted_iota(jnp.int32, (r, 1), 0) + i * r
    z = jnp.where(rowid < n, z, 0.0)
    z_ref[...] = z

    @pl.when(i == 0)
    def _():
        s1_ref[...] = jnp.zeros_like(s1_ref)
        s2_ref[...] = jnp.zeros_like(s2_ref)

    s1_ref[...] += jnp.sum(z, axis=0, keepdims=True)
    s2_ref[...] += jnp.sum(z * z, axis=0, keepdims=True)


@functools.partial(jax.jit, static_argnames=("n", "np_", "r"))
def _ca_call(acc, hs, dinv, b, n, np_, r):
    nb = np_ // r
    d = hs.shape[1]
    return pl.pallas_call(
        functools.partial(_ca_body, n, r),
        grid=(nb,),
        in_specs=[
            pl.BlockSpec((r, d), lambda i: (i, 0)),
            pl.BlockSpec((r, d), lambda i: (i, 0)),
            pl.BlockSpec((r, 1), lambda i: (i, 0)),
            pl.BlockSpec((1, d), lambda i: (0, 0)),
        ],
        out_specs=[
            pl.BlockSpec((r, d), lambda i: (i, 0)),
            pl.BlockSpec((1, d), lambda i: (0, 0)),
            pl.BlockSpec((1, d), lambda i: (0, 0)),
        ],
        out_shape=[
            jax.ShapeDtypeStruct((np_, d), jnp.float32),
            jax.ShapeDtypeStruct((1, d), jnp.float32),
            jax.ShapeDtypeStruct((1, d), jnp.float32),
        ],
    )(acc, hs, dinv, b)


def _cb_body(n, z_ref, s1_ref, s2_ref, g_ref, be_ref, w_ref, dinv_ref,
             hs_ref):
    m = s1_ref[...] * (1.0 / n)
    v = s2_ref[...] * (1.0 / n) - m * m
    sc = g_ref[...] * lax.rsqrt(v + 1e-5)
    y = (z_ref[...] - m) * sc + be_ref[...]
    h = jnp.dot(y, w_ref[...],
                preferred_element_type=jnp.float32,
                precision=lax.Precision.DEFAULT)
    hs_ref[...] = h * dinv_ref[...]


@functools.partial(jax.jit, static_argnames=("n", "np_", "r"))
def _cb_call(z, s1, s2, g, be, w, dinv, n, np_, r):
    nb = np_ // r
    d = z.shape[1]
    dout = w.shape[1]
    return pl.pallas_call(
        functools.partial(_cb_body, n),
        grid=(nb,),
        in_specs=[
            pl.BlockSpec((r, d), lambda i: (i, 0)),
            pl.BlockSpec((1, d), lambda i: (0, 0)),
            pl.BlockSpec((1, d), lambda i: (0, 0)),
            pl.BlockSpec((1, d), lambda i: (0, 0)),
            pl.BlockSpec((1, d), lambda i: (0, 0)),
            pl.BlockSpec((d, dout), lambda i: (0, 0)),
            pl.BlockSpec((r, 1), lambda i: (i, 0)),
        ],
        out_specs=pl.BlockSpec((r, dout), lambda i: (i, 0)),
        out_shape=jax.ShapeDtypeStruct((np_, dout), jnp.float32),
    )(z, s1, s2, g, be, w, dinv)


def _cf_body(n, z_ref, s1_ref, s2_ref, g_ref, be_ref, out_ref):
    m = s1_ref[...] * (1.0 / n)
    v = s2_ref[...] * (1.0 / n) - m * m
    sc = g_ref[...] * lax.rsqrt(v + 1e-5)
    out_ref[...] = (z_ref[...] - m) * sc + be_ref[...]


@functools.partial(jax.jit, static_argnames=("n", "r"))
def _cf_call(z, s1, s2, g, be, n, r):
    nb = n // r
    d = z.shape[1]
    return pl.pallas_call(
        functools.partial(_cf_body, n),
        grid=(nb,),
        in_specs=[
            pl.BlockSpec((r, d), lambda i: (i, 0)),
            pl.BlockSpec((1, d), lambda i: (0, 0)),
            pl.BlockSpec((1, d), lambda i: (0, 0)),
            pl.BlockSpec((1, d), lambda i: (0, 0)),
            pl.BlockSpec((1, d), lambda i: (0, 0)),
        ],
        out_specs=pl.BlockSpec((r, d), lambda i: (i, 0)),
        out_shape=jax.ShapeDtypeStruct((n, d), jnp.float32),
    )(z, s1, s2, g, be)


def kernel(x, edge_index, W1, b1, g1, be1, W2, b2, g2, be2, W3, b3, g3, be3,
           W4, b4, g4, be4):
    n, din = x.shape
    e = edge_index.shape[1]
    np_ = ((n + 1 + 127) // 128) * 128
    kc = -(-e // (NC * NS * CHUNK))
    kc = ((kc + 7) // 8) * 8
    kt = 2 * kc
    epad = kc * NC * NS * CHUNK
    r = np_ // 8

    pad_idx = jnp.full((epad - e,), n, jnp.int32)
    srcp = jnp.concatenate([edge_index[0], pad_idx]).reshape(-1, CHUNK)
    dstp = jnp.concatenate([edge_index[1], pad_idx]).reshape(-1, CHUNK)
    xp = jnp.pad(x, ((0, np_ - n), (0, 0)))

    deg = _deg_call(dstp, np_=np_, kc=kc).reshape(NC, np_)
    degt = deg.T

    hs, dinv = _c0_call(degt, xp, W1, np_=np_, r=r)
    layers = [(b1, g1, be1, W2), (b2, g2, be2, W3), (b3, g3, be3, W4)]
    for b, g, be, wnext in layers:
        d = hs.shape[1]
        acc = _scatter_call(hs, srcp, dstp, np_=np_, kt=kt, d=d)
        z, s1, s2 = _ca_call(acc, hs, dinv, b.reshape(1, d), n=n, np_=np_,
                             r=r)
        hs = _cb_call(z, s1, s2, g.reshape(1, d), be.reshape(1, d), wnext,
                      dinv, n=n, np_=np_, r=r)
    d = hs.shape[1]
    acc = _scatter_call(hs, srcp, dstp, np_=np_, kt=kt, d=d)
    z, s1, s2 = _ca_call(acc, hs, dinv, b4.reshape(1, d), n=n, np_=np_, r=r)
    return _cf_call(z, s1, s2, g4.reshape(1, d), be4.reshape(1, d), n=n,
                    r=n // 10)


# speedup vs baseline: 2.6941x; 1.0098x over previous
import functools

import jax
import jax.numpy as jnp
from jax import lax
from jax.experimental import pallas as pl
from jax.experimental.pallas import tpu as pltpu
from jax.experimental.pallas import tpu_sc as plsc

NC = 2
NS = 16
LANES = 16
CHUNK = 128

_MESH = dict(core_axis_name="c", subcore_axis_name="s", num_cores=NC,
             num_subcores=NS)


def _vzero(ref, n):
    for k in range(n // LANES):
        ref[pl.ds(k * LANES, LANES)] = jnp.zeros((LANES,), jnp.float32)
    if n % LANES:
        ref[pl.ds(n - LANES, LANES)] = jnp.zeros((LANES,), jnp.float32)


@functools.partial(jax.jit, static_argnames=("np_", "kc"))
def _deg_call(dstp, np_, kc):
    slab = np_ // NS
    mesh = plsc.VectorSubcoreMesh(**_MESH)

    @functools.partial(
        pl.kernel,
        out_type=jax.ShapeDtypeStruct((NC * np_,), jnp.float32),
        mesh=mesh,
        scratch_types=[
            pltpu.VMEM((kc, CHUNK), jnp.int32),
            pltpu.VMEM((CHUNK,), jnp.float32),
            pltpu.VMEM((slab,), jnp.float32),
            pltpu.VMEM_SHARED((np_,), jnp.float32),
            pltpu.SemaphoreType.DMA,
        ],
    )
    def k(dst_hbm, out_hbm, idx_v, ones_v, zbuf, hist, sem):
        c = lax.axis_index("c")
        s = lax.axis_index("s")
        wid = s * NC + c
        base = s * slab
        _vzero(zbuf, slab)
        for kk in range(CHUNK // LANES):
            ones_v[pl.ds(kk * LANES, LANES)] = jnp.ones((LANES,), jnp.float32)
        pltpu.sync_copy(zbuf, hist.at[pl.ds(base, slab)])
        plsc.subcore_barrier()
        pltpu.sync_copy(dst_hbm.at[pl.ds(wid * kc, kc)], idx_v)

        def body(j, carry):
            pltpu.sync_copy(ones_v, hist.at[idx_v.at[j]], add=True)
            return carry

        lax.fori_loop(0, kc, body, 0)
        plsc.subcore_barrier()
        pltpu.sync_copy(hist.at[pl.ds(base, slab)], zbuf)
        pltpu.sync_copy(zbuf, out_hbm.at[pl.ds(c * np_ + base, slab)])

    return k(dstp)


@functools.partial(jax.jit, static_argnames=("np_", "kt", "d"))
def _scatter_call(hs, srcp, dstp, np_, kt, d):
    slab = np_ // NS
    dh = d // 2
    mesh = plsc.VectorSubcoreMesh(**_MESH)

    @functools.partial(
        pl.kernel,
        out_type=jax.ShapeDtypeStruct((np_, d), jnp.float32),
        mesh=mesh,
        scratch_types=[
            pltpu.VMEM((2, CHUNK), jnp.int32),
            pltpu.VMEM((2, CHUNK), jnp.int32),
            pltpu.VMEM((kt, CHUNK), jnp.int32),
            pltpu.VMEM((CHUNK, dh), jnp.float32),
            pltpu.VMEM((CHUNK, dh), jnp.float32),
            pltpu.VMEM((CHUNK, dh), jnp.float32),
            pltpu.VMEM_SHARED((np_, dh), jnp.float32),
            pltpu.VMEM_SHARED((np_, dh), jnp.float32),
            pltpu.SemaphoreType.DMA,
            pltpu.SemaphoreType.DMA,
            pltpu.SemaphoreType.DMA,
            pltpu.SemaphoreType.DMA,
            pltpu.SemaphoreType.DMA,
            pltpu.SemaphoreType.DMA,
            pltpu.SemaphoreType.DMA,
            pltpu.SemaphoreType.DMA,
            pltpu.SemaphoreType.DMA,
        ],
        compiler_params=pltpu.CompilerParams(use_tc_tiling_on_sc=False),
    )
    def k(hs_hbm, src_hbm, dst_hbm, out_hbm, srca, srcb, didx, buf, buf1,
          buf2, hsh, acc, sem, sem1, isema, isemb, semz, semh1, semh2,
          sems1, sems2):
        c = lax.axis_index("c")
        s = lax.axis_index("s")
        base = s * slab
        cols = pl.ds(c * dh, dh)
        nfull, rem = slab // CHUNK, slab % CHUNK
        nf5 = nfull + (1 if rem else 0)
        bufs = [buf1, buf2]
        semh = [semh1, semh2]
        sems = [sems1, sems2]

        def run_pairs(hop1, hop2):
            for kk in range(nf5):
                if kk >= 2:
                    pltpu.make_async_copy(*hop2[kk - 2]).wait()
                pltpu.async_copy(*hop1[kk])
                if kk >= 1:
                    pltpu.make_async_copy(*hop1[kk - 1]).wait()
                    pltpu.async_copy(*hop2[kk - 1])
            pltpu.make_async_copy(*hop1[nf5 - 1]).wait()
            pltpu.async_copy(*hop2[nf5 - 1])
            pltpu.make_async_copy(*hop2[nf5 - 2]).wait()
            pltpu.make_async_copy(*hop2[nf5 - 1]).wait()

        def zb(rr, carry):
            for kk in range(dh // LANES):
                buf[rr, pl.ds(kk * LANES, LANES)] = jnp.zeros((LANES,),
                                                              jnp.float32)
            return carry

        lax.fori_loop(0, CHUNK, zb, 0)
        zdst = []
        for kk in range(nf5):
            nr = CHUNK if kk < nfull else rem
            zdst.append((buf.at[pl.ds(0, nr), :],
                         acc.at[pl.ds(base + kk * CHUNK, nr), :]))
            pltpu.async_copy(zdst[-1][0], zdst[-1][1], semz)
        cbase = s * kt
        pltpu.async_copy(dst_hbm.at[pl.ds(cbase, kt)], didx, isema)
        stage1, stage2 = [], []
        for kk in range(nf5):
            nr = CHUNK if kk < nfull else rem
            rows = pl.ds(base + kk * CHUNK, nr)
            bx = bufs[kk % 2].at[pl.ds(0, nr), :]
            stage1.append((hs_hbm.at[rows, cols], bx, semh[kk % 2]))
            stage2.append((bx, hsh.at[rows, :], sems[kk % 2]))
        run_pairs(stage1, stage2)
        for z in zdst:
            pltpu.make_async_copy(z[0], z[1], semz).wait()
        pltpu.make_async_copy(src_hbm.at[pl.ds(0, kt)], didx, isema).wait()
        plsc.subcore_barrier()

        dummy = hs_hbm.at[pl.ds(0, CHUNK), cols]
        idummy = src_hbm.at[pl.ds(0, 2)]
        pltpu.sync_copy(src_hbm.at[pl.ds(cbase, 2)], srca)
        pltpu.async_copy(hsh.at[srca.at[0]], buf, sem)

        def half(j0, cur, nxt, isem_n, ioff):
            pltpu.async_copy(src_hbm.at[pl.ds(ioff, 2)], nxt, isem_n)
            pltpu.make_async_copy(dummy, buf, sem).wait()
            pltpu.async_copy(hsh.at[cur.at[1]], buf1, sem1)
            pltpu.sync_copy(buf, acc.at[didx.at[j0]], add=True)
            pltpu.make_async_copy(dummy, buf1, sem1).wait()
            pltpu.make_async_copy(idummy, nxt, isem_n).wait()
            pltpu.async_copy(hsh.at[nxt.at[0]], buf, sem)
            pltpu.sync_copy(buf1, acc.at[didx.at[j0 + 1]], add=True)

        def body(m, carry):
            j0 = m * 4
            half(j0, srca, srcb, isemb, cbase + j0 + 2)
            half(j0 + 2, srcb, srca, isema,
                 cbase + jnp.minimum(j0 + 4, kt - 2))
            return carry

        lax.fori_loop(0, kt // 4, body, 0)
        pltpu.make_async_copy(dummy, buf, sem).wait()
        plsc.subcore_barrier()
        out1, out2 = [], []
        for kk in range(nf5):
            nr = CHUNK if kk < nfull else rem
            rows = pl.ds(base + kk * CHUNK, nr)
            bx = bufs[kk % 2].at[pl.ds(0, nr), :]
            out1.append((acc.at[rows, :], bx, semh[kk % 2]))
            out2.append((bx, out_hbm.at[rows, cols], sems[kk % 2]))
        run_pairs(out1, out2)

    return k(hs, srcp, dstp)


def _c0_body(deg_ref, x_ref, w_ref, hs_ref, dinv_ref):
    d = deg_ref[...]
    dv = lax.rsqrt(d[:, 0:1] + d[:, 1:2] + 1.0)
    h = jnp.dot(x_ref[...], w_ref[...],
                preferred_element_type=jnp.float32,
                precision=lax.Precision.DEFAULT)
    hs_ref[...] = h * dv
    dinv_ref[...] = dv


@functools.partial(jax.jit, static_argnames=("np_", "r"))
def _c0_call(degt, xp, w1, np_, r):
    nb = np_ // r
    din = xp.shape[1]
    return pl.pallas_call(
        _c0_body,
        grid=(nb,),
        in_specs=[
            pl.BlockSpec((r, 2), lambda i: (i, 0)),
            pl.BlockSpec((r, din), lambda i: (i, 0)),
            pl.BlockSpec((din, din), lambda i: (0, 0)),
        ],
        out_specs=[
            pl.BlockSpec((r, din), lambda i: (i, 0)),
            pl.BlockSpec((r, 1), lambda i: (i, 0)),
        ],
        out_shape=[
            jax.ShapeDtypeStruct((np_, din), jnp.float32),
            jax.ShapeDtypeStruct((np_, 1), jnp.float32),
        ],
    )(degt, xp, w1)


def _cab_body(n, r, acc_ref, hs_ref, dinv_ref, b_ref, g_ref, be_ref,
              w_ref, hsn_ref, z_scr, s1, s2):
    ph = pl.program_id(0)
    i = pl.program_id(1)

    @pl.when(ph == 0)
    def _():
        pre = dinv_ref[...] * (acc_ref[...] + hs_ref[...]) + b_ref[...]
        z = jnp.maximum(pre, 0.0)
        rowid = lax.broadcasted_iota(jnp.int32, (r, 1), 0) + i * r
        z = jnp.where(rowid < n, z, 0.0)
        z_scr[pl.ds(i * r, r), :] = z

        @pl.when(i == 0)
        def _():
            s1[...] = jnp.zeros_like(s1)
            s2[...] = jnp.zeros_like(s2)

        s1[...] += jnp.sum(z, axis=0, keepdims=True)
        s2[...] += jnp.sum(z * z, axis=0, keepdims=True)

    @pl.when(ph == 1)
    def _():
        m = s1[...] * (1.0 / n)
        v = s2[...] * (1.0 / n) - m * m
        sc = g_ref[...] * lax.rsqrt(v + 1e-5)
        y = (z_scr[pl.ds(i * r, r), :] - m) * sc + be_ref[...]
        h = jnp.dot(y, w_ref[...],
                    preferred_element_type=jnp.float32,
                    precision=lax.Precision.DEFAULT)
        hsn_ref[...] = h * dinv_ref[...]


@functools.partial(jax.jit, static_argnames=("n", "np_", "r"))
def _cab_call(acc, hs, dinv, b, g, be, w, n, np_, r):
    nb = np_ // r
    d = hs.shape[1]
    dout = w.shape[1]
    once = lambda ph, i: ((1 - ph) * i, 0)
    return pl.pallas_call(
        functools.partial(_cab_body, n, r),
        grid=(2, nb),
        in_specs=[
            pl.BlockSpec((r, d), once),
            pl.BlockSpec((r, d), once),
            pl.BlockSpec((r, 1), lambda ph, i: (i, 0)),
            pl.BlockSpec((1, d), lambda ph, i: (0, 0)),
            pl.BlockSpec((1, d), lambda ph, i: (0, 0)),
            pl.BlockSpec((1, d), lambda ph, i: (0, 0)),
            pl.BlockSpec((d, dout), lambda ph, i: (0, 0)),
        ],
        out_specs=pl.BlockSpec((r, dout), lambda ph, i: (i, 0)),
        out_shape=jax.ShapeDtypeStruct((np_, dout), jnp.float32),
        scratch_shapes=[
            pltpu.VMEM((np_, d), jnp.float32),
            pltpu.VMEM((1, d), jnp.float32),
            pltpu.VMEM((1, d), jnp.float32),
        ],
    )(acc, hs, dinv, b, g, be, w)


def _caf_body(n, r, acc_ref, hs_ref, dinv_ref, b_ref, g_ref, be_ref,
              out_ref, z_scr, s1, s2):
    ph = pl.program_id(0)
    i = pl.program_id(1)

    @pl.when(ph == 0)
    def _():
        pre = dinv_ref[...] * (acc_ref[...] + hs_ref[...]) + b_ref[...]
        z = jnp.maximum(pre, 0.0)
        rowid = lax.broadcasted_iota(jnp.int32, (r, 1), 0) + i * r
        z = jnp.where(rowid < n, z, 0.0)
        z_scr[pl.ds(i * r, r), :] = z

        @pl.when(i == 0)
        def _():
            s1[...] = jnp.zeros_like(s1)
            s2[...] = jnp.zeros_like(s2)

        s1[...] += jnp.sum(z, axis=0, keepdims=True)
        s2[...] += jnp.sum(z * z, axis=0, keepdims=True)

    @pl.when(ph == 1)
    def _():
        m = s1[...] * (1.0 / n)
        v = s2[...] * (1.0 / n) - m * m
        sc = g_ref[...] * lax.rsqrt(v + 1e-5)
        out_ref[...] = (z_scr[pl.ds(i * r, r), :] - m) * sc + be_ref[...]


@functools.partial(jax.jit, static_argnames=("n", "np_", "r"))
def _caf_call(acc, hs, dinv, b, g, be, n, np_, r):
    nb = np_ // r
    d = hs.shape[1]
    once = lambda ph, i: ((1 - ph) * i, 0)
    return pl.pallas_call(
        functools.partial(_caf_body, n, r),
        grid=(2, nb),
        in_specs=[
            pl.BlockSpec((r, d), once),
            pl.BlockSpec((r, d), once),
            pl.BlockSpec((r, 1), lambda ph, i: (i, 0)),
            pl.BlockSpec((1, d), lambda ph, i: (0, 0)),
            pl.BlockSpec((1, d), lambda ph, i: (0, 0)),
            pl.BlockSpec((1, d), lambda ph, i: (0, 0)),
        ],
        out_specs=pl.BlockSpec((r, d), lambda ph, i: (i, 0)),
        out_shape=jax.ShapeDtypeStruct((n, d), jnp.float32),
        scratch_shapes=[
            pltpu.VMEM((np_, d), jnp.float32),
            pltpu.VMEM((1, d), jnp.float32),
            pltpu.VMEM((1, d), jnp.float32),
        ],
    )(acc, hs, dinv, b, g, be)


def kernel(x, edge_index, W1, b1, g1, be1, W2, b2, g2, be2, W3, b3, g3, be3,
           W4, b4, g4, be4):
    n, din = x.shape
    e = edge_index.shape[1]
    np_ = ((n + 1 + 127) // 128) * 128
    kc = -(-e // (NC * NS * CHUNK))
    kc = ((kc + 7) // 8) * 8
    kt = 2 * kc
    epad = kc * NC * NS * CHUNK
    r = np_ // 8

    pad_idx = jnp.full((epad - e,), n, jnp.int32)
    srcp = jnp.concatenate([edge_index[0], pad_idx]).reshape(-1, CHUNK)
    dstp = jnp.concatenate([edge_index[1], pad_idx]).reshape(-1, CHUNK)
    xp = jnp.pad(x, ((0, np_ - n), (0, 0)))

    deg = _deg_call(dstp, np_=np_, kc=kc).reshape(NC, np_)
    degt = deg.T

    hs, dinv = _c0_call(degt, xp, W1, np_=np_, r=r)
    layers = [(b1, g1, be1, W2), (b2, g2, be2, W3), (b3, g3, be3, W4)]
    for b, g, be, wnext in layers:
        d = hs.shape[1]
        acc = _scatter_call(hs, srcp, dstp, np_=np_, kt=kt, d=d)
        hs = _cab_call(acc, hs, dinv, b.reshape(1, d), g.reshape(1, d),
                       be.reshape(1, d), wnext, n=n, np_=np_, r=r)
    d = hs.shape[1]
    acc = _scatter_call(hs, srcp, dstp, np_=np_, kt=kt, d=d)
    return _caf_call(acc, hs, dinv, b4.reshape(1, d), g4.reshape(1, d),
                     be4.reshape(1, d), n=n, np_=np_, r=r)
